# Initial kernel scaffold; baseline (speedup 1.0000x reference)
import jax
import jax.numpy as jnp
from jax.experimental import pallas as pl


def kernel(node_features, edge_label_index):
    raise NotImplementedError("write your pallas kernel here")


# speedup vs baseline: 4.5367x; 4.5367x over previous
import functools

import jax
import jax.numpy as jnp
from jax import lax
from jax.experimental import pallas as pl
from jax.experimental.pallas import tpu as pltpu
from jax.experimental.pallas import tpu_sc as plsc

N_NODES = 10000
DIMS = 128
N_EDGES = 320000
R = 2.0
T = 1.0

NC = 2
NS = 16
NW = NC * NS
E_W = N_EDGES // NW
C = 80
N_CHUNKS = E_W // C
L = 16
GROUPS = C // L
NSEG = DIMS // L

_mesh = plsc.VectorSubcoreMesh(core_axis_name="c", subcore_axis_name="s")


@functools.partial(
    pl.kernel,
    out_type=jax.ShapeDtypeStruct((N_EDGES,), jnp.float32),
    mesh=_mesh,
    scratch_types=[
        pltpu.VMEM((E_W,), jnp.int32),
        pltpu.VMEM((E_W,), jnp.int32),
        pltpu.VMEM((E_W,), jnp.float32),
        pltpu.VMEM((C, DIMS), jnp.float32),
        pltpu.VMEM((C, DIMS), jnp.float32),
        pltpu.VMEM((C, DIMS), jnp.float32),
        pltpu.VMEM((C, DIMS), jnp.float32),
        pltpu.SemaphoreType.DMA,
        pltpu.SemaphoreType.DMA,
        pltpu.SemaphoreType.DMA,
        pltpu.SemaphoreType.DMA,
    ],
    compiler_params=pltpu.CompilerParams(needs_layout_passes=False),
)
def _lp_decode(table, eidx, out, idx_u, idx_v, out_v, ru0, ru1, rv0, rv1,
               su0, su1, sv0, sv1):
    wid = lax.axis_index("s") * NC + lax.axis_index("c")
    base = pl.multiple_of(wid * E_W, 8)

    rows_u = (ru0, ru1)
    rows_v = (rv0, rv1)
    sem_u = (su0, su1)
    sem_v = (sv0, sv1)

    pltpu.sync_copy(eidx.at[pl.ds(base, E_W)], idx_u)
    pltpu.sync_copy(eidx.at[pl.ds(N_EDGES + base, E_W)], idx_v)

    def issue(ch, b):
        off = pl.multiple_of(ch * C, 8)
        pltpu.async_copy(table.at[idx_u.at[pl.ds(off, C)]], rows_u[b], sem_u[b])
        pltpu.async_copy(table.at[idx_v.at[pl.ds(off, C)]], rows_v[b], sem_v[b])

    def wait(ch, b):
        off = pl.multiple_of(ch * C, 8)
        pltpu.make_async_copy(table.at[idx_u.at[pl.ds(off, C)]], rows_u[b], sem_u[b]).wait()
        pltpu.make_async_copy(table.at[idx_v.at[pl.ds(off, C)]], rows_v[b], sem_v[b]).wait()

    row_ids = lax.iota(jnp.int32, L)

    def compute(ch, b):
        ru = rows_u[b]
        rv = rows_v[b]

        def group_body(g, _):
            e0 = g * L
            tot = jnp.zeros((L,), jnp.float32)
            for i in range(L):
                e = e0 + i
                acc = None
                for k in range(NSEG):
                    du = ru[e, pl.ds(k * L, L)] - rv[e, pl.ds(k * L, L)]
                    sq = du * du
                    acc = sq if acc is None else acc + sq
                s = jnp.sum(acc)
                tot = jnp.where(row_ids == i, s, tot)
            probs = 1.0 / (jnp.exp((tot - R) / T) + 1.0)
            out_v[pl.ds(ch * C + e0, L)] = probs
            return 0

        lax.fori_loop(0, GROUPS, group_body, 0)

    issue(0, 0)
    issue(1, 1)

    def pair_body(p, _):
        for b in range(2):
            ch = p * 2 + b
            wait(ch, b)
            compute(ch, b)

            @pl.when(ch + 2 < N_CHUNKS)
            def _():
                issue(ch + 2, b)

        return 0

    lax.fori_loop(0, (N_CHUNKS - 1) // 2, pair_body, 0)
    wait(N_CHUNKS - 1, 0)
    compute(N_CHUNKS - 1, 0)

    pltpu.sync_copy(out_v, out.at[pl.ds(base, E_W)])


def kernel(node_features, edge_label_index):
    return _lp_decode(node_features, edge_label_index.reshape(-1))


# speedup vs baseline: 5.1060x; 1.1255x over previous
import functools

import jax
import jax.numpy as jnp
from jax import lax
from jax.experimental import pallas as pl
from jax.experimental.pallas import tpu as pltpu
from jax.experimental.pallas import tpu_sc as plsc

N_NODES = 10000
DIMS = 128
N_EDGES = 320000
R = 2.0
T = 1.0

NC = 2
NS = 16
NW = NC * NS
E_W = N_EDGES // NW
C = 40
N_CHUNKS = E_W // C
L = 16
GROUPS = C // L
assert E_W % C == 0 and N_CHUNKS % 2 == 0 and C >= L and C % 8 == 0
NSEG = DIMS // L

_mesh = plsc.VectorSubcoreMesh(core_axis_name="c", subcore_axis_name="s")


@functools.partial(
    pl.kernel,
    out_type=jax.ShapeDtypeStruct((N_EDGES,), jnp.float32),
    mesh=_mesh,
    scratch_types=[
        pltpu.VMEM((E_W,), jnp.int32),
        pltpu.VMEM((E_W,), jnp.int32),
        pltpu.VMEM((C,), jnp.float32),
        pltpu.VMEM((C,), jnp.float32),
        pltpu.VMEM((C, DIMS), jnp.float32),
        pltpu.VMEM((C, DIMS), jnp.float32),
        pltpu.VMEM((C, DIMS), jnp.float32),
        pltpu.VMEM((C, DIMS), jnp.float32),
        pltpu.SemaphoreType.DMA,
        pltpu.SemaphoreType.DMA,
        pltpu.SemaphoreType.DMA,
        pltpu.SemaphoreType.DMA,
        pltpu.SemaphoreType.DMA,
        pltpu.SemaphoreType.DMA,
    ],
    compiler_params=pltpu.CompilerParams(needs_layout_passes=False),
)
def _lp_decode(table, eidx, out, idx_u, idx_v, ob0, ob1, ru0, ru1, rv0, rv1,
               su0, su1, sv0, sv1, so0, so1):
    wid = lax.axis_index("s") * NC + lax.axis_index("c")
    base = pl.multiple_of(wid * E_W, 8)

    rows_u = (ru0, ru1)
    rows_v = (rv0, rv1)
    outb = (ob0, ob1)
    sem_u = (su0, su1)
    sem_v = (sv0, sv1)
    sem_o = (so0, so1)

    pltpu.sync_copy(eidx.at[pl.ds(base, E_W)], idx_u)
    pltpu.sync_copy(eidx.at[pl.ds(N_EDGES + base, E_W)], idx_v)

    def issue(ch, b):
        off = pl.multiple_of(ch * C, 8)
        pltpu.async_copy(table.at[idx_u.at[pl.ds(off, C)]], rows_u[b], sem_u[b])
        pltpu.async_copy(table.at[idx_v.at[pl.ds(off, C)]], rows_v[b], sem_v[b])

    def wait(ch, b):
        off = pl.multiple_of(ch * C, 8)
        pltpu.make_async_copy(table.at[idx_u.at[pl.ds(off, C)]], rows_u[b], sem_u[b]).wait()
        pltpu.make_async_copy(table.at[idx_v.at[pl.ds(off, C)]], rows_v[b], sem_v[b]).wait()

    row_ids = lax.iota(jnp.int32, L)

    def compute(ch, b):
        ru = rows_u[b]
        rv = rows_v[b]

        def group_body(g, _):
            e0 = g * L
            tot = jnp.zeros((L,), jnp.float32)
            for i in range(L):
                e = e0 + i
                acc = None
                for k in range(NSEG):
                    du = ru[e, pl.ds(k * L, L)] - rv[e, pl.ds(k * L, L)]
                    sq = du * du
                    acc = sq if acc is None else acc + sq
                s = jnp.sum(acc)
                tot = jnp.where(row_ids == i, s, tot)
            probs = 1.0 / (jnp.exp((tot - R) / T) + 1.0)
            outb[b][pl.ds(e0, L)] = probs
            return 0

        lax.fori_loop(0, GROUPS, group_body, 0)
        if C % L != 0:
            def last_group(e0):
                tot = jnp.zeros((L,), jnp.float32)
                for i in range(L):
                    e = e0 + i
                    acc = None
                    for k in range(NSEG):
                        du = ru[e, pl.ds(k * L, L)] - rv[e, pl.ds(k * L, L)]
                        sq = du * du
                        acc = sq if acc is None else acc + sq
                    s = jnp.sum(acc)
                    tot = jnp.where(row_ids == i, s, tot)
                probs = 1.0 / (jnp.exp((tot - R) / T) + 1.0)
                outb[b][pl.ds(e0, L)] = probs
            last_group(C - L)

    def write_out(ch, b):
        off = pl.multiple_of(base + ch * C, 8)
        pltpu.async_copy(outb[b], out.at[pl.ds(off, C)], sem_o[b])

    def wait_out(ch, b):
        off = pl.multiple_of(base + ch * C, 8)
        pltpu.make_async_copy(outb[b], out.at[pl.ds(off, C)], sem_o[b]).wait()

    issue(0, 0)
    issue(1, 1)

    def pair_body(p, _):
        for b in range(2):
            ch = p * 2 + b
            wait(ch, b)

            @pl.when(ch >= 2)
            def _():
                wait_out(ch - 2, b)

            compute(ch, b)
            write_out(ch, b)

            @pl.when(ch + 2 < N_CHUNKS)
            def _():
                issue(ch + 2, b)

        return 0

    lax.fori_loop(0, N_CHUNKS // 2, pair_body, 0)
    wait_out(N_CHUNKS - 2, 0)
    wait_out(N_CHUNKS - 1, 1)


def kernel(node_features, edge_label_index):
    return _lp_decode(node_features, edge_label_index.reshape(-1))
